# baseline (speedup 1.0000x reference)
import jax
import jax.numpy as jnp
from jax import lax
from jax.experimental import pallas as pl
from jax.experimental.pallas import tpu as pltpu
from jax.experimental.pallas import tpu_sc as plsc

NC = 2
NS = 16
NW = NC * NS
LANES = 16

B, C, H, W = 4, 96, 512, 512
P = B * H * W
PW = P // NW
CHUNK = 2048
NCHUNK = PW // CHUNK
HW = H * W
LOG2_HW = 18


def _sc_body(z_hbm, l_hbm, lp_hbm, cond_hbm, out_hbm,
             l_v, lp_v, c_v, ig_v, ib_v, g_v, b_v, acc_v,
             sem_in0, sem_in1, sem_g0, sem_g1):
    wid = lax.axis_index("s") * NC + lax.axis_index("c")
    base = wid * PW
    iota = lax.iota(jnp.int32, LANES)
    sem_in = (sem_in0, sem_in1)
    sem_g = (sem_g0, sem_g1)

    def start_in(k):
        u = k % 2
        off = base + k * CHUNK
        return (
            pltpu.async_copy(l_hbm.at[pl.ds(off, CHUNK)], l_v[u], sem_in[u]),
            pltpu.async_copy(lp_hbm.at[pl.ds(off, CHUNK)], lp_v[u], sem_in[u]),
            pltpu.async_copy(cond_hbm.at[pl.ds(off, CHUNK)], c_v[u], sem_in[u]),
        )

    def do_idx(k):
        u = k % 2
        off = base + k * CHUNK

        @plsc.parallel_loop(0, CHUNK // LANES, unroll=8)
        def _(j):
            s = pl.ds(j * LANES, LANES)
            lv = l_v[u][s]
            lpv = lp_v[u][s]
            cv = c_v[u][s]
            p = (off + j * LANES) + iota
            bb = lax.shift_right_logical(p, LOG2_HW)
            pos = p + bb * ((C - 1) * HW)
            ig = pos + lax.shift_left(lv, LOG2_HW)
            ib = pos + lax.shift_left(lpv, LOG2_HW)
            ib = jnp.where(cv != 0, ib, ig)
            ig_v[u][s] = ig
            ib_v[u][s] = ib

    def start_g(k):
        u = k % 2
        return (
            pltpu.async_copy(z_hbm.at[ig_v[u]], g_v[u], sem_g[u]),
            pltpu.async_copy(z_hbm.at[ib_v[u]], b_v[u], sem_g[u]),
        )

    def do_acc(k, acc):
        u = k % 2

        @plsc.parallel_loop(0, CHUNK // LANES, unroll=8, carry=acc)
        def acc2(j, a):
            s = pl.ds(j * LANES, LANES)
            return a + (g_v[u][s] - b_v[u][s])

        return acc2

    acc = jnp.zeros((LANES,), jnp.float32)
    ins = [None] * (NCHUNK + 1)
    gs = [None] * (NCHUNK + 1)
    ins[0] = start_in(0)
    for d in ins[0]:
        d.wait()
    do_idx(0)
    ins[1] = start_in(1)
    gs[0] = start_g(0)
    for k in range(NCHUNK):
        if k + 1 < NCHUNK:
            for d in ins[k + 1]:
                d.wait()
            do_idx(k + 1)
            if k + 2 < NCHUNK:
                ins[k + 2] = start_in(k + 2)
            gs[k + 1] = start_g(k + 1)
        for d in gs[k]:
            d.wait()
        acc = do_acc(k, acc)

    acc_v[...] = acc
    pltpu.sync_copy(acc_v, out_hbm.at[wid])


def _reduce_body(x_ref, o_ref):
    o_ref[0, 0] = jnp.sum(x_ref[...])


def _phys_view(x):
    s = x.shape[:-2]
    n = len(s)
    x6 = x.reshape(*s, H // 8, 8, W // 128, 128)
    return jnp.transpose(x6, tuple(range(n)) + (n, n + 2, n + 1, n + 3)
                         ).reshape(-1)


@jax.jit
def kernel(z, condition, l, l_prime):
    z_flat = _phys_view(z)
    l_i = _phys_view(l.astype(jnp.int32))
    lp_i = _phys_view(l_prime.astype(jnp.int32))
    c_i = _phys_view(condition.astype(jnp.int32))

    mesh = plsc.VectorSubcoreMesh(
        core_axis_name="c", subcore_axis_name="s",
        num_cores=NC, num_subcores=NS)
    partials = pl.kernel(
        _sc_body,
        out_type=jax.ShapeDtypeStruct((NW, LANES), jnp.float32),
        mesh=mesh,
        scratch_types=[
            [pltpu.VMEM((CHUNK,), jnp.int32)] * 2,
            [pltpu.VMEM((CHUNK,), jnp.int32)] * 2,
            [pltpu.VMEM((CHUNK,), jnp.int32)] * 2,
            [pltpu.VMEM((CHUNK,), jnp.int32)] * 2,
            [pltpu.VMEM((CHUNK,), jnp.int32)] * 2,
            [pltpu.VMEM((CHUNK,), jnp.float32)] * 2,
            [pltpu.VMEM((CHUNK,), jnp.float32)] * 2,
            pltpu.VMEM((LANES,), jnp.float32),
            pltpu.SemaphoreType.DMA,
            pltpu.SemaphoreType.DMA,
            pltpu.SemaphoreType.DMA,
            pltpu.SemaphoreType.DMA,
        ],
    )(z_flat, l_i, lp_i, c_i)

    loss = pl.pallas_call(
        _reduce_body,
        out_shape=jax.ShapeDtypeStruct((1, 1), jnp.float32),
        out_specs=pl.BlockSpec(memory_space=pltpu.SMEM),
    )(partials.reshape(4, 128))
    return loss[0, 0]


# speedup vs baseline: 1.0170x; 1.0170x over previous
import jax
import jax.numpy as jnp
from jax import lax
from jax.experimental import pallas as pl
from jax.experimental.pallas import tpu as pltpu
from jax.experimental.pallas import tpu_sc as plsc

NC = 2
NS = 16
NW = NC * NS
LANES = 16

B, C, H, W = 4, 96, 512, 512
P = B * H * W
PW = P // NW
CHUNK = 4096
NCHUNK = PW // CHUNK
HW = H * W
LOG2_HW = 18


def _sc_body(z_hbm, l_hbm, lp_hbm, cond_hbm, out_hbm,
             l_v, lp_v, c_v, ig_v, ib_v, g_v, b_v, acc_v,
             sem_in0, sem_in1, sem_g0, sem_g1):
    wid = lax.axis_index("s") * NC + lax.axis_index("c")
    base = wid * PW
    iota = lax.iota(jnp.int32, LANES)
    sem_in = (sem_in0, sem_in1)
    sem_g = (sem_g0, sem_g1)

    def start_in(k):
        u = k % 2
        off = base + k * CHUNK
        return (
            pltpu.async_copy(l_hbm.at[pl.ds(off, CHUNK)], l_v[u], sem_in[u]),
            pltpu.async_copy(lp_hbm.at[pl.ds(off, CHUNK)], lp_v[u], sem_in[u]),
            pltpu.async_copy(cond_hbm.at[pl.ds(off, CHUNK)], c_v[u], sem_in[u]),
        )

    def do_idx(k):
        u = k % 2
        off = base + k * CHUNK

        @plsc.parallel_loop(0, CHUNK // LANES, unroll=8)
        def _(j):
            s = pl.ds(j * LANES, LANES)
            lv = l_v[u][s]
            lpv = lp_v[u][s]
            cv = c_v[u][s]
            p = (off + j * LANES) + iota
            bb = lax.shift_right_logical(p, LOG2_HW)
            pos = p + bb * ((C - 1) * HW)
            ig = pos + lax.shift_left(lv, LOG2_HW)
            ib = pos + lax.shift_left(lpv, LOG2_HW)
            ib = jnp.where(cv != 0, ib, ig)
            ig_v[u][s] = ig
            ib_v[u][s] = ib

    def start_g(k):
        u = k % 2
        return (
            pltpu.async_copy(z_hbm.at[ig_v[u]], g_v[u], sem_g[u]),
            pltpu.async_copy(z_hbm.at[ib_v[u]], b_v[u], sem_g[u]),
        )

    def do_acc(k, acc):
        u = k % 2

        @plsc.parallel_loop(0, CHUNK // LANES, unroll=8, carry=acc)
        def acc2(j, a):
            s = pl.ds(j * LANES, LANES)
            return a + (g_v[u][s] - b_v[u][s])

        return acc2

    acc = jnp.zeros((LANES,), jnp.float32)
    ins = [None] * (NCHUNK + 1)
    gs = [None] * (NCHUNK + 1)
    ins[0] = start_in(0)
    for d in ins[0]:
        d.wait()
    do_idx(0)
    ins[1] = start_in(1)
    gs[0] = start_g(0)
    for k in range(NCHUNK):
        if k + 1 < NCHUNK:
            for d in ins[k + 1]:
                d.wait()
            do_idx(k + 1)
            if k + 2 < NCHUNK:
                ins[k + 2] = start_in(k + 2)
            gs[k + 1] = start_g(k + 1)
        for d in gs[k]:
            d.wait()
        acc = do_acc(k, acc)

    acc_v[...] = acc
    pltpu.sync_copy(acc_v, out_hbm.at[wid])


def _reduce_body(x_ref, o_ref):
    o_ref[0, 0] = jnp.sum(x_ref[...])


def _phys_view(x):
    s = x.shape[:-2]
    n = len(s)
    x6 = x.reshape(*s, H // 8, 8, W // 128, 128)
    return jnp.transpose(x6, tuple(range(n)) + (n, n + 2, n + 1, n + 3)
                         ).reshape(-1)


@jax.jit
def kernel(z, condition, l, l_prime):
    z_flat = _phys_view(z)
    l_i = _phys_view(l.astype(jnp.int32))
    lp_i = _phys_view(l_prime.astype(jnp.int32))
    c_i = _phys_view(condition.astype(jnp.int32))

    mesh = plsc.VectorSubcoreMesh(
        core_axis_name="c", subcore_axis_name="s",
        num_cores=NC, num_subcores=NS)
    partials = pl.kernel(
        _sc_body,
        out_type=jax.ShapeDtypeStruct((NW, LANES), jnp.float32),
        mesh=mesh,
        scratch_types=[
            [pltpu.VMEM((CHUNK,), jnp.int32)] * 2,
            [pltpu.VMEM((CHUNK,), jnp.int32)] * 2,
            [pltpu.VMEM((CHUNK,), jnp.int32)] * 2,
            [pltpu.VMEM((CHUNK,), jnp.int32)] * 2,
            [pltpu.VMEM((CHUNK,), jnp.int32)] * 2,
            [pltpu.VMEM((CHUNK,), jnp.float32)] * 2,
            [pltpu.VMEM((CHUNK,), jnp.float32)] * 2,
            pltpu.VMEM((LANES,), jnp.float32),
            pltpu.SemaphoreType.DMA,
            pltpu.SemaphoreType.DMA,
            pltpu.SemaphoreType.DMA,
            pltpu.SemaphoreType.DMA,
        ],
    )(z_flat, l_i, lp_i, c_i)

    loss = pl.pallas_call(
        _reduce_body,
        out_shape=jax.ShapeDtypeStruct((1, 1), jnp.float32),
        out_specs=pl.BlockSpec(memory_space=pltpu.SMEM),
    )(partials.reshape(4, 128))
    return loss[0, 0]
